# baseline (speedup 1.0000x reference)
import functools

import jax
import jax.numpy as jnp
from jax import lax
from jax.experimental import pallas as pl
from jax.experimental.pallas import tpu as pltpu
from jax.experimental.pallas import tpu_sc as plsc

N_NODES = 10000
DIM = 128

NC = 2
NS = 16
NW = NC * NS

CHUNK = 128
ZC = 32
N_PAD = 10240
ROW_BLK = 400


def _matmul_y2(x, w2):
    def body(x_ref, w_ref, o_ref):
        o_ref[...] = jnp.dot(x_ref[...], w_ref[...],
                             preferred_element_type=jnp.float32)

    grid = N_NODES // ROW_BLK
    return pl.pallas_call(
        body,
        grid=(grid,),
        in_specs=[
            pl.BlockSpec((ROW_BLK, DIM), lambda i: (i, 0)),
            pl.BlockSpec((DIM, DIM), lambda i: (0, 0)),
        ],
        out_specs=pl.BlockSpec((ROW_BLK, DIM), lambda i: (i, 0)),
        out_shape=jax.ShapeDtypeStruct((N_NODES, DIM), jnp.float32),
    )(x, w2)


NBUF = 2
IRING = 4
UNROLL = 4


def _make_sc_scatter(n_chunks):
    rows_per_tile = N_PAD // NS
    mesh = plsc.VectorSubcoreMesh(core_axis_name="c", subcore_axis_name="s")

    @functools.partial(
        pl.kernel,
        mesh=mesh,
        out_type=jax.ShapeDtypeStruct((NC, N_PAD, DIM), jnp.float32),
        scratch_types=[
            pltpu.VMEM((ZC, DIM), jnp.float32),
            [pltpu.VMEM((CHUNK,), jnp.int32)] * IRING,
            [pltpu.VMEM((CHUNK,), jnp.int32)] * IRING,
            [pltpu.VMEM((CHUNK, DIM), jnp.float32)] * NBUF,
            pltpu.VMEM_SHARED((N_PAD, DIM), jnp.float32),
            [pltpu.SemaphoreType.DMA] * IRING,
            [pltpu.SemaphoreType.DMA] * IRING,
            [pltpu.SemaphoreType.DMA] * NBUF,
        ],
    )
    def sc_kernel(src_hbm, dst_hbm, y2_hbm, out_hbm,
                  zbuf, sring, dring, rows, acc, jsem, isem, gsem):
        cid = lax.axis_index("c")
        sid = lax.axis_index("s")
        wid = sid * NC + cid
        e0 = wid * n_chunks * CHUNK

        def src_load(g, d):
            return pltpu.make_async_copy(
                src_hbm.at[pl.ds(e0 + g * CHUNK, CHUNK)], sring[d],
                jsem[d])

        def dst_load(g, d):
            return pltpu.make_async_copy(
                dst_hbm.at[pl.ds(e0 + g * CHUNK, CHUNK)], dring[d],
                isem[d])

        def gather(slot, b):
            return pltpu.make_async_copy(
                y2_hbm.at[sring[slot]], rows[b], gsem[b])

        for d in range(IRING):
            src_load(d, d).start()
            dst_load(d, d).start()

        def zrow(i, carry):
            for j in range(DIM // 16):
                zbuf[i, pl.ds(j * 16, 16)] = jnp.zeros((16,), jnp.float32)
            return carry
        lax.fori_loop(0, ZC, zrow, 0)

        def zcopy(i, carry):
            pltpu.sync_copy(
                zbuf, acc.at[pl.ds(sid * rows_per_tile + i * ZC, ZC)])
            return carry
        lax.fori_loop(0, rows_per_tile // ZC, zcopy, 0)
        plsc.subcore_barrier()

        for b in range(NBUF):
            src_load(b, b).wait()
            gather(b, b).start()

        def group_body(g2, carry):
            gbase = g2 * UNROLL
            for u in range(UNROLL):
                b = u % NBUF
                d = u % IRING
                g = gbase + u
                gather(u, b).wait()
                dst_load(g, d).wait()
                pltpu.sync_copy(rows[b], acc.at[dring[d]], add=True)
                d2 = (u + NBUF) % IRING
                src_load(g + NBUF, d2).wait()
                gather(d2, b).start()
                src_load(g + IRING, d).start()
                dst_load(g + IRING, d).start()
            return carry
        lax.fori_loop(0, (n_chunks - NBUF) // UNROLL, group_body, 0)

        for u in range(NBUF):
            g = n_chunks - NBUF + u
            b = g % NBUF
            d = g % IRING
            gather(d, b).wait()
            dst_load(g, d).wait()
            pltpu.sync_copy(rows[b], acc.at[dring[d]], add=True)
        plsc.subcore_barrier()

        lo = sid * rows_per_tile
        pltpu.sync_copy(acc.at[pl.ds(lo, rows_per_tile)],
                        out_hbm.at[cid, pl.ds(lo, rows_per_tile)])

    return sc_kernel


def _final(x, w1, partials):
    def body(x_ref, w_ref, p_ref, o_ref):
        acc = jnp.dot(x_ref[...], w_ref[...],
                      preferred_element_type=jnp.float32)
        acc = acc + p_ref[0] + p_ref[1]
        o_ref[...] = jnp.maximum(acc, 0.0)

    grid = N_NODES // ROW_BLK
    return pl.pallas_call(
        body,
        grid=(grid,),
        in_specs=[
            pl.BlockSpec((ROW_BLK, DIM), lambda i: (i, 0)),
            pl.BlockSpec((DIM, DIM), lambda i: (0, 0)),
            pl.BlockSpec((NC, ROW_BLK, DIM), lambda i: (0, i, 0)),
        ],
        out_specs=pl.BlockSpec((ROW_BLK, DIM), lambda i: (i, 0)),
        out_shape=jax.ShapeDtypeStruct((N_NODES, DIM), jnp.float32),
    )(x, w1, partials)


def kernel(x, edge_index, W1, W2):
    src = edge_index[0].astype(jnp.int32)
    dst = edge_index[1].astype(jnp.int32)
    n_edges = src.shape[0]
    per = NW * CHUNK
    n_chunks = -(-n_edges // per)
    while n_chunks % UNROLL != NBUF or n_chunks < NBUF + UNROLL:
        n_chunks += 1
    e_pad = n_chunks * per
    pad = e_pad - n_edges
    if pad:
        pad_dst = N_NODES + jnp.arange(pad, dtype=jnp.int32) % (N_PAD - N_NODES)
        src = jnp.concatenate([src, jnp.zeros((pad,), jnp.int32)])
        dst = jnp.concatenate([dst, pad_dst])
    tail = jnp.zeros((UNROLL * CHUNK,), jnp.int32)
    src = jnp.concatenate([src, tail])
    dst = jnp.concatenate([dst, tail])

    y2 = _matmul_y2(x, W2)
    partials = _make_sc_scatter(n_chunks)(src, dst, y2)
    return _final(x, W1, partials)


# speedup vs baseline: 1.3738x; 1.3738x over previous
import functools

import jax
import jax.numpy as jnp
from jax import lax
from jax.experimental import pallas as pl
from jax.experimental.pallas import tpu as pltpu
from jax.experimental.pallas import tpu_sc as plsc

N_NODES = 10000
DIM = 128

NC = 2
NS = 16
NW = NC * NS

CHUNK = 128
ZC = 32
N_PAD = 10240
ROW_BLK = 400


def _matmul_y2(x, w2):
    def body(x_ref, w_ref, o_ref):
        o_ref[...] = jnp.dot(x_ref[...], w_ref[...],
                             preferred_element_type=jnp.float32)

    grid = N_NODES // ROW_BLK
    return pl.pallas_call(
        body,
        grid=(grid,),
        in_specs=[
            pl.BlockSpec((ROW_BLK, DIM), lambda i: (i, 0)),
            pl.BlockSpec((DIM, DIM), lambda i: (0, 0)),
        ],
        out_specs=pl.BlockSpec((ROW_BLK, DIM), lambda i: (i, 0)),
        out_shape=jax.ShapeDtypeStruct((N_NODES, DIM), jnp.float32),
    )(x, w2)


K = 2


def _make_sc_scatter(n_chunks):
    rows_per_tile = N_PAD // NS
    mesh = plsc.VectorSubcoreMesh(core_axis_name="c", subcore_axis_name="s")

    @functools.partial(
        pl.kernel,
        mesh=mesh,
        out_type=jax.ShapeDtypeStruct((NC, N_PAD, DIM), jnp.float32),
        scratch_types=[
            pltpu.VMEM((ZC, DIM), jnp.float32),
            [pltpu.VMEM((CHUNK,), jnp.int32)] * K,
            [pltpu.VMEM((CHUNK,), jnp.int32)] * K,
            [pltpu.VMEM((CHUNK, DIM), jnp.float32)] * K,
            pltpu.VMEM_SHARED((N_PAD, DIM), jnp.float32),
            pltpu.SemaphoreType.DMA,
            pltpu.SemaphoreType.DMA,
        ],
    )
    def sc_kernel(src_hbm, dst_hbm, y2_hbm, out_hbm,
                  zbuf, sidx, didx, rows, acc, isem, gsem):
        cid = lax.axis_index("c")
        sid = lax.axis_index("s")
        wid = sid * NC + cid
        e0 = wid * n_chunks * CHUNK

        def zrow(i, carry):
            for j in range(DIM // 16):
                zbuf[i, pl.ds(j * 16, 16)] = jnp.zeros((16,), jnp.float32)
            return carry
        lax.fori_loop(0, ZC, zrow, 0)

        def zcopy(i, carry):
            pltpu.sync_copy(
                zbuf, acc.at[pl.ds(sid * rows_per_tile + i * ZC, ZC)])
            return carry
        lax.fori_loop(0, rows_per_tile // ZC, zcopy, 0)
        plsc.subcore_barrier()

        def group_body(g2, carry):
            base = e0 + g2 * (K * CHUNK)
            il = []
            for k in range(K):
                il.append(pltpu.make_async_copy(
                    src_hbm.at[pl.ds(base + k * CHUNK, CHUNK)],
                    sidx[k], isem))
                il.append(pltpu.make_async_copy(
                    dst_hbm.at[pl.ds(base + (n_chunks * NW * CHUNK)
                                     + k * CHUNK, CHUNK)],
                    didx[k], isem))
            for d in il:
                d.start()
            for d in il:
                d.wait()
            gl = [pltpu.make_async_copy(y2_hbm.at[sidx[k]], rows[k], gsem)
                  for k in range(K)]
            for d in gl:
                d.start()
            for d in gl:
                d.wait()
            for k in range(K):
                pltpu.sync_copy(rows[k], acc.at[didx[k]], add=True)
            return carry
        lax.fori_loop(0, n_chunks // K, group_body, 0)
        plsc.subcore_barrier()

        lo = sid * rows_per_tile
        pltpu.sync_copy(acc.at[pl.ds(lo, rows_per_tile)],
                        out_hbm.at[cid, pl.ds(lo, rows_per_tile)])

    return sc_kernel


def _final(x, w1, partials):
    def body(x_ref, w_ref, p_ref, o_ref):
        acc = jnp.dot(x_ref[...], w_ref[...],
                      preferred_element_type=jnp.float32)
        acc = acc + p_ref[0] + p_ref[1]
        o_ref[...] = jnp.maximum(acc, 0.0)

    grid = N_NODES // ROW_BLK
    return pl.pallas_call(
        body,
        grid=(grid,),
        in_specs=[
            pl.BlockSpec((ROW_BLK, DIM), lambda i: (i, 0)),
            pl.BlockSpec((DIM, DIM), lambda i: (0, 0)),
            pl.BlockSpec((NC, ROW_BLK, DIM), lambda i: (0, i, 0)),
        ],
        out_specs=pl.BlockSpec((ROW_BLK, DIM), lambda i: (i, 0)),
        out_shape=jax.ShapeDtypeStruct((N_NODES, DIM), jnp.float32),
    )(x, w1, partials)


def kernel(x, edge_index, W1, W2):
    src = edge_index[0].astype(jnp.int32)
    dst = edge_index[1].astype(jnp.int32)
    n_edges = src.shape[0]
    per = NW * CHUNK
    n_chunks = -(-n_edges // per)
    n_chunks = -(-n_chunks // K) * K
    e_pad = n_chunks * per
    pad = e_pad - n_edges
    if pad:
        pad_dst = N_NODES + jnp.arange(pad, dtype=jnp.int32) % (N_PAD - N_NODES)
        src = jnp.concatenate([src, jnp.zeros((pad,), jnp.int32)])
        dst = jnp.concatenate([dst, pad_dst])
    sd = jnp.concatenate([src, dst])

    y2 = _matmul_y2(x, W2)
    partials = _make_sc_scatter(n_chunks)(sd, sd, y2)
    return _final(x, W1, partials)
